# baseline (device time: 80315 ns/iter reference)
import jax
import jax.numpy as jnp
from jax import lax
from jax.experimental import pallas as pl
from jax.experimental.pallas import tpu as pltpu

N_DEV = 4
N_COLS_GLOBAL = 4096
EPS = 1e-5
CHUNK = 512


def kernel(x, gamma):
    m, n = x.shape
    nc = m // CHUNK
    g2 = gamma.reshape(1, n)

    def body(x_ref, g_ref, out_ref, xv_ref, part_ref, comm_ref, inv_ref,
             send_sems, recv_sems):
        c = pl.program_id(0)
        my = lax.axis_index("i")

        @pl.when(c == 0)
        def _():
            barrier_sem = pltpu.get_barrier_semaphore()
            for j in range(1, N_DEV):
                peer = (my + j) % N_DEV
                pl.semaphore_signal(
                    barrier_sem, inc=1,
                    device_id=(peer,), device_id_type=pl.DeviceIdType.MESH,
                )
            pl.semaphore_wait(barrier_sem, N_DEV - 1)

        ones = jnp.ones((n, 1), jnp.float32)

        @pl.when(c < nc)
        def _():
            xv = x_ref[...]
            xv_ref[pl.ds(c * CHUNK, CHUNK), :] = xv
            x2 = xv * xv
            part_ref[pl.ds(c * CHUNK, CHUNK), :] = jax.lax.dot_general(
                x2, ones, (((1,), (0,)), ((), ())),
                preferred_element_type=jnp.float32,
            )

        def mk_rdma(j):
            peer = (my + j) % N_DEV
            return pltpu.make_async_remote_copy(
                src_ref=part_ref,
                dst_ref=comm_ref.at[j - 1],
                send_sem=send_sems.at[j - 1],
                recv_sem=recv_sems.at[j - 1],
                device_id=(peer,),
                device_id_type=pl.DeviceIdType.MESH,
            )

        @pl.when(c == nc - 1)
        def _():
            for j in range(1, N_DEV):
                mk_rdma(j).start()

        @pl.when(c == nc)
        def _():
            for j in range(1, N_DEV):
                r = mk_rdma(j)
                r.wait_recv()
                r.wait_send()
            total = part_ref[...] + comm_ref[0] + comm_ref[1] + comm_ref[2]
            inv_ref[...] = lax.rsqrt(total * (1.0 / N_COLS_GLOBAL) + EPS)

        @pl.when(c >= nc)
        def _():
            cc = c - nc
            xv = xv_ref[pl.ds(cc * CHUNK, CHUNK), :]
            invg = jax.lax.dot_general(
                inv_ref[pl.ds(cc * CHUNK, CHUNK), :], g_ref[...],
                (((1,), (0,)), ((), ())),
                preferred_element_type=jnp.float32,
            )
            out_ref[...] = xv * invg

    return pl.pallas_call(
        body,
        grid=(2 * nc,),
        out_shape=jax.ShapeDtypeStruct((m, n), jnp.float32),
        in_specs=[
            pl.BlockSpec((CHUNK, n), lambda c: (jnp.minimum(c, nc - 1), 0)),
            pl.BlockSpec((1, n), lambda c: (0, 0)),
        ],
        out_specs=pl.BlockSpec((CHUNK, n), lambda c: (jnp.maximum(c - nc, 0), 0)),
        scratch_shapes=[
            pltpu.VMEM((m, n), jnp.float32),
            pltpu.VMEM((m, 1), jnp.float32),
            pltpu.VMEM((3, m, 1), jnp.float32),
            pltpu.VMEM((m, 1), jnp.float32),
            pltpu.SemaphoreType.DMA((3,)),
            pltpu.SemaphoreType.DMA((3,)),
        ],
        compiler_params=pltpu.CompilerParams(
            collective_id=0,
            vmem_limit_bytes=64 * 1024 * 1024,
        ),
    )(x, g2)


# device time: 29231 ns/iter; 2.7476x vs baseline; 2.7476x over previous
import os

import jax
import jax.numpy as jnp
from jax import lax
from jax.experimental import pallas as pl
from jax.experimental.pallas import tpu as pltpu

ABLATE = int(os.environ.get("ABLATE_COMM", "0"))
ABLATE_BARRIER = ABLATE == 1
ABLATE_RDMA = ABLATE in (1, 2)

N_DEV = 4
N_COLS_GLOBAL = 4096
EPS = 1e-5
CHUNK = 1024


def kernel(x, gamma):
    m, n = x.shape
    nc = m // CHUNK
    g2 = gamma.reshape(1, n)

    def body(x_ref, g_ref, out_ref, xv_ref, part_ref, dense_ref,
             comm_ref, inv_ref, send_sems, recv_sems):
        c = pl.program_id(0)
        my = lax.axis_index("i")

        barrier_sem = pltpu.get_barrier_semaphore()

        @pl.when((c == 0) & jnp.bool_(not ABLATE_BARRIER))
        def _():
            for j in range(1, N_DEV):
                peer = (my + j) % N_DEV
                pl.semaphore_signal(
                    barrier_sem, inc=1,
                    device_id=(peer,), device_id_type=pl.DeviceIdType.MESH,
                )

        ones = jnp.ones((n, 1), jnp.float32)

        @pl.when(c < nc)
        def _():
            with jax.named_scope("partial"):
                xv = x_ref[...]
                xv_ref[pl.ds(c * CHUNK, CHUNK), :] = xv
                part_ref[pl.ds(c * CHUNK, CHUNK), :] = jnp.sum(
                    xv * xv, axis=1, keepdims=True
                )

        hd = (m // 128) // 2
        hm = m // 2

        def mk_rdma(h, j):
            peer = (my + j) % N_DEV
            sl = pl.ds(h * hd, hd)
            return pltpu.make_async_remote_copy(
                src_ref=dense_ref.at[sl, :],
                dst_ref=comm_ref.at[j - 1, sl, :],
                send_sem=send_sems.at[h, j - 1],
                recv_sem=recv_sems.at[h, j - 1],
                device_id=(peer,),
                device_id_type=pl.DeviceIdType.MESH,
            )

        for h in range(2):
            @pl.when((c == (h + 1) * (nc // 2) - 1) & jnp.bool_(not ABLATE_RDMA))
            def _(h=h):
                dense_ref[pl.ds(h * hd, hd), :] = part_ref[
                    pl.ds(h * hm, hm), :
                ].reshape(hd, 128)
                if h == 0:
                    pl.semaphore_wait(barrier_sem, N_DEV - 1)
                for j in range(1, N_DEV):
                    mk_rdma(h, j).start()

            @pl.when(c == nc + h * (nc // 2))
            def _(h=h):
                with jax.named_scope(f"exchange_wait{h}"):
                    if not ABLATE_RDMA:
                        for j in range(1, N_DEV):
                            r = mk_rdma(h, j)
                            r.wait_recv()
                            r.wait_send()
                with jax.named_scope(f"reduce_inv{h}"):
                    sl = pl.ds(h * hd, hd)
                    total = (dense_ref[sl, :] + comm_ref[0, sl, :]
                             + comm_ref[1, sl, :] + comm_ref[2, sl, :])
                    inv = lax.rsqrt(total * (1.0 / N_COLS_GLOBAL) + EPS)
                    xpand = jnp.broadcast_to(
                        inv[:, None, :], (hd, 128, 128)
                    ).reshape(hm, 128)
                    lane = lax.broadcasted_iota(jnp.int32, (hm, 128), 1)
                    row = lax.broadcasted_iota(jnp.int32, (hm, 128), 0)
                    msk = (lane == row % 128).astype(jnp.float32)
                    inv_ref[pl.ds(h * hm, hm), :] = lax.dot_general(
                        xpand * msk, jnp.ones((128, 1), jnp.float32),
                        (((1,), (0,)), ((), ())),
                        preferred_element_type=jnp.float32,
                    )

        @pl.when(c >= nc)
        def _():
            with jax.named_scope("normalize"):
                cc = c - nc
                invg = jax.lax.dot_general(
                    inv_ref[pl.ds(cc * CHUNK, CHUNK), :], g_ref[...],
                    (((1,), (0,)), ((), ())),
                    preferred_element_type=jnp.float32,
                )
                out_ref[...] = xv_ref[pl.ds(cc * CHUNK, CHUNK), :] * invg

    return pl.pallas_call(
        body,
        grid=(2 * nc,),
        out_shape=jax.ShapeDtypeStruct((m, n), jnp.float32),
        in_specs=[
            pl.BlockSpec((CHUNK, n), lambda c: (jnp.minimum(c, nc - 1), 0)),
            pl.BlockSpec((1, n), lambda c: (0, 0)),
        ],
        out_specs=pl.BlockSpec((CHUNK, n), lambda c: (jnp.maximum(c - nc, 0), 0)),
        scratch_shapes=[
            pltpu.VMEM((m, n), jnp.float32),
            pltpu.VMEM((m, 1), jnp.float32),
            pltpu.VMEM((m // 128, 128), jnp.float32),
            pltpu.VMEM((3, m // 128, 128), jnp.float32),
            pltpu.VMEM((m, 1), jnp.float32),
            pltpu.SemaphoreType.DMA((2, 3)),
            pltpu.SemaphoreType.DMA((2, 3)),
        ],
        compiler_params=pltpu.CompilerParams(
            collective_id=0,
            vmem_limit_bytes=100 * 1024 * 1024,
        ),
    )(x, g2)


# device time: 29025 ns/iter; 2.7671x vs baseline; 1.0071x over previous
import os

import jax
import jax.numpy as jnp
from jax import lax
from jax.experimental import pallas as pl
from jax.experimental.pallas import tpu as pltpu

ABLATE = int(os.environ.get("ABLATE_COMM", "0"))
ABLATE_BARRIER = ABLATE == 1
ABLATE_RDMA = ABLATE in (1, 2)

N_DEV = 4
N_COLS_GLOBAL = 4096
EPS = 1e-5
CHUNK = 1024


def kernel(x, gamma):
    m, n = x.shape
    nc = m // CHUNK
    g2 = gamma.reshape(1, n)

    def body(x_ref, g_ref, out_ref, xv_ref, part_ref, dense_ref,
             comm_ref, inv_ref, send_sems, recv_sems):
        c = pl.program_id(0)
        my = lax.axis_index("i")

        barrier_sem = pltpu.get_barrier_semaphore()

        @pl.when((c == 0) & jnp.bool_(not ABLATE_BARRIER))
        def _():
            for j in range(1, N_DEV):
                peer = (my + j) % N_DEV
                pl.semaphore_signal(
                    barrier_sem, inc=1,
                    device_id=(peer,), device_id_type=pl.DeviceIdType.MESH,
                )

        ones = jnp.ones((n, 1), jnp.float32)

        @pl.when(c < nc)
        def _():
            with jax.named_scope("partial"):
                xv = x_ref[...]
                xv_ref[pl.ds(c * CHUNK, CHUNK), :] = xv
                part_ref[pl.ds(c * CHUNK, CHUNK), :] = jnp.sum(
                    xv * xv, axis=1, keepdims=True
                )

        hd = (m // 128) // 2
        hm = m // 2

        def mk_rdma(h, j):
            peer = (my + j) % N_DEV
            sl = pl.ds(h * hd, hd)
            return pltpu.make_async_remote_copy(
                src_ref=dense_ref.at[sl, :],
                dst_ref=comm_ref.at[j - 1, sl, :],
                send_sem=send_sems.at[h, j - 1],
                recv_sem=recv_sems.at[h, j - 1],
                device_id=(peer,),
                device_id_type=pl.DeviceIdType.MESH,
            )

        for h in range(2):
            @pl.when((c == (h + 1) * (nc // 2) - 1) & jnp.bool_(not ABLATE_RDMA))
            def _(h=h):
                dense_ref[pl.ds(h * hd, hd), :] = part_ref[
                    pl.ds(h * hm, hm), :
                ].reshape(hd, 128)
                if h == 0:
                    pl.semaphore_wait(barrier_sem, N_DEV - 1)
                for j in range(1, N_DEV):
                    mk_rdma(h, j).start()

            @pl.when(c == nc + h * (nc // 2))
            def _(h=h):
                with jax.named_scope(f"exchange_wait{h}"):
                    if not ABLATE_RDMA:
                        for j in range(1, N_DEV):
                            r = mk_rdma(h, j)
                            r.wait_recv()
                            r.wait_send()
                with jax.named_scope(f"reduce_inv{h}"):
                    sl = pl.ds(h * hd, hd)
                    total = (dense_ref[sl, :] + comm_ref[0, sl, :]
                             + comm_ref[1, sl, :] + comm_ref[2, sl, :])
                    inv = lax.rsqrt(total * (1.0 / N_COLS_GLOBAL) + EPS)
                    xpand = jnp.broadcast_to(
                        inv[:, None, :], (hd, 128, 128)
                    ).reshape(hm, 128)
                    lane = lax.broadcasted_iota(jnp.int32, (hm, 128), 1)
                    row = lax.broadcasted_iota(jnp.int32, (hm, 128), 0)
                    msk = (lane == row % 128).astype(jnp.float32)
                    inv_ref[pl.ds(h * hm, hm), :] = lax.dot_general(
                        xpand * msk, jnp.ones((128, 1), jnp.float32),
                        (((1,), (0,)), ((), ())),
                        preferred_element_type=jnp.float32,
                    )

        @pl.when(c >= nc)
        def _():
            with jax.named_scope("normalize"):
                cc = c - nc
                inv = inv_ref[pl.ds(cc * CHUNK, CHUNK), :]
                out_ref[...] = xv_ref[pl.ds(cc * CHUNK, CHUNK), :] * (
                    inv * g_ref[...]
                )

    return pl.pallas_call(
        body,
        grid=(2 * nc,),
        out_shape=jax.ShapeDtypeStruct((m, n), jnp.float32),
        in_specs=[
            pl.BlockSpec((CHUNK, n), lambda c: (jnp.minimum(c, nc - 1), 0)),
            pl.BlockSpec((1, n), lambda c: (0, 0)),
        ],
        out_specs=pl.BlockSpec((CHUNK, n), lambda c: (jnp.maximum(c - nc, 0), 0)),
        scratch_shapes=[
            pltpu.VMEM((m, n), jnp.float32),
            pltpu.VMEM((m, 1), jnp.float32),
            pltpu.VMEM((m // 128, 128), jnp.float32),
            pltpu.VMEM((3, m // 128, 128), jnp.float32),
            pltpu.VMEM((m, 1), jnp.float32),
            pltpu.SemaphoreType.DMA((2, 3)),
            pltpu.SemaphoreType.DMA((2, 3)),
        ],
        compiler_params=pltpu.CompilerParams(
            collective_id=0,
            vmem_limit_bytes=100 * 1024 * 1024,
        ),
    )(x, g2)
